# baseline (speedup 1.0000x reference)
import functools

import jax
import jax.numpy as jnp
from jax import lax
from jax.experimental import pallas as pl
from jax.experimental.pallas import tpu as pltpu
from jax.experimental.pallas import tpu_sc as plsc

GROUPS = 4
V = 1024
B = 1024
D = 256
BSC = 768
EG = 8
SR = 16
NSL = D // SR
NC = 2
NS = 16
L = 16
NW = NC * NS
BPW = B // NW
WPW = BSC // NW
NDMA = (WPW // EG) * NSL

_mesh = plsc.VectorSubcoreMesh(core_axis_name="c", subcore_axis_name="s")


@functools.partial(
    pl.kernel,
    mesh=_mesh,
    out_type=(
        jax.ShapeDtypeStruct((BSC, D, D), jnp.float32),
        jax.ShapeDtypeStruct((B, D), jnp.float32),
    ),
    scratch_types=[
        pltpu.VMEM((L,), jnp.int32),
        pltpu.VMEM((L,), jnp.int32),
        pltpu.VMEM((BPW,), jnp.int32),
        pltpu.VMEM((BPW,), jnp.int32),
        pltpu.VMEM((BPW,), jnp.int32),
        pltpu.VMEM((BPW,), jnp.int32),
        pltpu.VMEM((BPW, D), jnp.float32),
        pltpu.VMEM((EG, SR, D), jnp.float32),
        pltpu.VMEM((EG, SR, D), jnp.float32),
        pltpu.VMEM((EG, SR, D), jnp.float32),
        pltpu.SemaphoreType.DMA,
        pltpu.SemaphoreType.DMA,
        pltpu.SemaphoreType.DMA,
        pltpu.SemaphoreType.DMA,
        pltpu.SemaphoreType.DMA,
        pltpu.SemaphoreType.DMA,
        pltpu.SemaphoreType.DMA,
    ],
)
def _sc_lookup(ld, selflat, wtab, btab, outw, outb,
               z_v, g_v, sidx_v, widx_v, sidxw_v, widxw_v, bbuf,
               wbuf0, wbuf1, wbuf2,
               usem, gs0, gs1, gs2, ps0, ps1, ps2):
    wid = lax.axis_index("s") * NC + lax.axis_index("c")
    base = pl.multiple_of(wid * BPW, BPW)
    basew = pl.multiple_of(wid * WPW, EG)
    iota = lax.iota(jnp.int32, L)

    z_v[...] = iota * 0
    pltpu.async_copy(ld.at[z_v], g_v, usem).wait()
    g = g_v[...]

    for h in range(BPW // L):
        sidxw_v[pl.ds(h * L, L)] = (basew + h * L + iota) * GROUPS + g
        sidx_v[pl.ds(h * L, L)] = (base + h * L + iota) * GROUPS + g
    pltpu.async_copy(selflat.at[sidxw_v], widxw_v, usem).wait()

    bufs = (wbuf0, wbuf1, wbuf2)
    gsems = (gs0, gs1, gs2)
    psems = (ps0, ps1, ps2)

    def g_desc(d, b):
        o = pl.multiple_of((d // NSL) * EG, EG)
        c = pl.multiple_of((d % NSL) * SR, SR)
        return pltpu.make_async_copy(
            wtab.at[widxw_v.at[pl.ds(o, EG)], pl.ds(c, SR)], bufs[b], gsems[b])

    def p_desc(d, b):
        o = pl.multiple_of((d // NSL) * EG, EG)
        c = pl.multiple_of((d % NSL) * SR, SR)
        return pltpu.make_async_copy(
            bufs[b], outw.at[pl.ds(basew + o, EG), pl.ds(c, SR)], psems[b])

    g_desc(0, 0).start()
    g_desc(1, 1).start()

    pltpu.async_copy(selflat.at[sidx_v], widx_v, usem).wait()
    b_gather = pltpu.make_async_copy(btab.at[widx_v], bbuf, usem)
    b_gather.start()

    def step(i, carry):
        for k in range(3):
            d = i * 3 + k
            g_desc(d, k).wait()
            p_desc(d, k).start()
            dn = d + 2
            bn = (k + 2) % 3

            @pl.when(dn < NDMA)
            def _():
                @pl.when(d >= 1)
                def _():
                    p_desc(d - 1, bn).wait()

                g_desc(dn, bn).start()
        return carry

    lax.fori_loop(0, NDMA // 3, step, 0)
    b_gather.wait()
    pltpu.sync_copy(bbuf, outb.at[pl.ds(base, BPW)])
    p_desc(NDMA - 3, (NDMA - 3) % 3).wait()
    p_desc(NDMA - 2, (NDMA - 2) % 3).wait()
    p_desc(NDMA - 1, (NDMA - 1) % 3).wait()


def _tc_body(widx_ref, w_ref, out_ref):
    out_ref[...] = w_ref[...]


_tc_gather = pl.pallas_call(
    _tc_body,
    grid_spec=pltpu.PrefetchScalarGridSpec(
        num_scalar_prefetch=1,
        grid=(B - BSC,),
        in_specs=[pl.BlockSpec((1, D, D), lambda i, widx: (widx[i], 0, 0))],
        out_specs=pl.BlockSpec((1, D, D), lambda i, widx: (i, 0, 0)),
    ),
    out_shape=jax.ShapeDtypeStruct((B - BSC, D, D), jnp.float32),
)


def kernel(layer_index, selector, W0, b0, layers_distribution):
    ld = lax.dynamic_slice_in_dim(layers_distribution, layer_index, 1)
    selflat = selector.reshape(B * GROUPS)
    outw_sc, outb = _sc_lookup(ld, selflat, W0, b0)
    widx_tc = selector[BSC:, ld[0]]
    outw_tc = _tc_gather(widx_tc, W0)
    outw = jnp.concatenate([outw_sc, outw_tc], axis=0)
    return (outw, outb)


# speedup vs baseline: 2.0296x; 2.0296x over previous
import functools

import jax
import jax.numpy as jnp
from jax import lax
from jax.experimental import pallas as pl
from jax.experimental.pallas import tpu as pltpu
from jax.experimental.pallas import tpu_sc as plsc

GROUPS = 4
V = 1024
B = 1024
D = 256
EG = 8
SR = 16
NSL = D // SR
NC = 2
NS = 16
L = 16
NW = NC * NS
BPW = B // NW
NDMA = (BPW // EG) * NSL

_mesh = plsc.VectorSubcoreMesh(core_axis_name="c", subcore_axis_name="s")


@functools.partial(
    pl.kernel,
    mesh=_mesh,
    out_type=(
        jax.ShapeDtypeStruct((B, D, D), jnp.float32),
        jax.ShapeDtypeStruct((B, D), jnp.float32),
    ),
    scratch_types=[
        pltpu.VMEM((L,), jnp.int32),
        pltpu.VMEM((L,), jnp.int32),
        pltpu.VMEM((BPW,), jnp.int32),
        pltpu.VMEM((BPW,), jnp.int32),
        pltpu.VMEM((BPW, D), jnp.float32),
        pltpu.VMEM((EG, SR, D), jnp.float32),
        pltpu.VMEM((EG, SR, D), jnp.float32),
        pltpu.VMEM((EG, SR, D), jnp.float32),
        pltpu.SemaphoreType.DMA,
        pltpu.SemaphoreType.DMA,
        pltpu.SemaphoreType.DMA,
        pltpu.SemaphoreType.DMA,
        pltpu.SemaphoreType.DMA,
        pltpu.SemaphoreType.DMA,
        pltpu.SemaphoreType.DMA,
    ],
)
def _sc_lookup(ld, selflat, wtab, btab, outw, outb,
               z_v, g_v, sidx_v, widx_v, bbuf, wbuf0, wbuf1, wbuf2,
               usem, gs0, gs1, gs2, ps0, ps1, ps2):
    wid = lax.axis_index("s") * NC + lax.axis_index("c")
    base = pl.multiple_of(wid * BPW, BPW)
    iota = lax.iota(jnp.int32, L)

    z_v[...] = iota * 0
    pltpu.async_copy(ld.at[z_v], g_v, usem).wait()
    g = g_v[...]

    for h in range(BPW // L):
        sidx_v[pl.ds(h * L, L)] = (base + h * L + iota) * GROUPS + g
    pltpu.async_copy(selflat.at[sidx_v], widx_v, usem).wait()

    b_gather = pltpu.make_async_copy(btab.at[widx_v], bbuf, usem)
    b_gather.start()

    bufs = (wbuf0, wbuf1, wbuf2)
    gsems = (gs0, gs1, gs2)
    psems = (ps0, ps1, ps2)

    def g_desc(d, b):
        o = pl.multiple_of((d // NSL) * EG, EG)
        c = pl.multiple_of((d % NSL) * SR, SR)
        return pltpu.make_async_copy(
            wtab.at[widx_v.at[pl.ds(o, EG)], pl.ds(c, SR)], bufs[b], gsems[b])

    def p_desc(d, b):
        o = pl.multiple_of((d // NSL) * EG, EG)
        c = pl.multiple_of((d % NSL) * SR, SR)
        return pltpu.make_async_copy(
            bufs[b], outw.at[pl.ds(base + o, EG), pl.ds(c, SR)], psems[b])

    g_desc(0, 0).start()
    g_desc(1, 1).start()

    def step(i, carry):
        for k in range(3):
            d = i * 3 + k
            g_desc(d, k).wait()
            p_desc(d, k).start()
            dn = d + 2
            bn = (k + 2) % 3

            @pl.when(dn < NDMA)
            def _():
                @pl.when(d >= 1)
                def _():
                    p_desc(d - 1, bn).wait()

                g_desc(dn, bn).start()
        return carry

    lax.fori_loop(0, (NDMA - 1) // 3, step, 0)
    g_desc(NDMA - 1, 0).wait()
    p_desc(NDMA - 1, 0).start()
    b_gather.wait()
    pltpu.sync_copy(bbuf, outb.at[pl.ds(base, BPW)])
    p_desc(NDMA - 3, 1).wait()
    p_desc(NDMA - 2, 2).wait()
    p_desc(NDMA - 1, 0).wait()


def kernel(layer_index, selector, W0, b0, layers_distribution):
    ld = lax.dynamic_slice_in_dim(layers_distribution, layer_index, 1)
    selflat = selector.reshape(B * GROUPS)
    outw, outb = _sc_lookup(ld, selflat, W0, b0)
    return (outw, outb)
